# Initial kernel scaffold; baseline (speedup 1.0000x reference)
import jax
import jax.numpy as jnp
from jax.experimental import pallas as pl


def kernel(down_w0, down_b0, down_w, down_b, pool_w, up_w, up_b, up_w_last, up_b_last, w1, b1, w2, b2, wd, bd, x0, A0, x1, A1):
    raise NotImplementedError("write your pallas kernel here")


# speedup vs baseline: 1.0236x; 1.0236x over previous
import math

import jax
import jax.numpy as jnp
from jax import lax
from jax.experimental import pallas as pl
from jax.experimental.pallas import tpu as pltpu

TOTAL_LATENT = 97
PAD_LATENT = 128
K_SORT = 30
C1, C2 = 16, 32
KW2 = 5
OUT_DIM = 10
F_IN = 8
H = 48
RATIOS = [0.9, 0.7, 0.6, 0.5]
DEPTH = len(RATIOS)
KP = (K_SORT - 2) // 2 + 1
LOUT = KP - KW2 + 1

N_G0, N_G1 = 1536, 1408


def _ladder(n):
    out = [n]
    for r in RATIOS:
        out.append(int(math.ceil(r * out[-1])))
    return out


LAD0 = _ladder(N_G0)
LAD1 = _ladder(N_G1)
SPAD = [N_G0] + [(max(a, b) + 7) // 8 * 8 for a, b in zip(LAD0[1:], LAD1[1:])]


def _unet_kernel(a_ref, x_ref, dw0_ref, db0_ref, dw_ref, db_ref, pw_ref,
                 uw_ref, ub_ref, uwl_ref, ubl_ref,
                 w1_ref, b1_ref, w2_ref, b2_ref, wd_ref, bd_ref, o_ref):
    f32 = jnp.float32
    pid = pl.program_id(0)
    is0 = pid == 0

    def sel(a, b):
        return jnp.where(is0, a, b)

    nv0 = sel(LAD0[0], LAD1[0])
    kv = [sel(LAD0[i + 1], LAD1[i + 1]) for i in range(DEPTH)]

    _eyes, _ltm, _cols = {}, {}, {}

    def eye(n):
        if n not in _eyes:
            r = lax.broadcasted_iota(jnp.int32, (n, n), 0)
            c = lax.broadcasted_iota(jnp.int32, (n, n), 1)
            _eyes[n] = jnp.where(r == c, 1.0, 0.0)
        return _eyes[n]

    def lt_mask(n):
        if n not in _ltm:
            r = lax.broadcasted_iota(jnp.int32, (n, n), 0)
            c = lax.broadcasted_iota(jnp.int32, (n, n), 1)
            _ltm[n] = c < r
        return _ltm[n]

    def col_iota(n):
        if n not in _cols:
            _cols[n] = lax.broadcasted_iota(jnp.int32, (n, 1), 0)
        return _cols[n]

    def rowvec(v_col, n):
        diag = eye(n) * jnp.broadcast_to(v_col, (n, n))
        return jnp.dot(jnp.ones((1, n), f32), diag, preferred_element_type=f32)

    def ranks(s_col, n):
        s_row = jnp.broadcast_to(rowvec(s_col, n), (n, n))
        s_cb = jnp.broadcast_to(s_col, (n, n))
        before = jnp.where((s_row > s_cb) | ((s_row == s_cb) & lt_mask(n)),
                           1.0, 0.0)
        return jnp.sum(before, axis=1, keepdims=True)

    def gcn(A, x, W, b, relu, n):
        ahat = A + 2.0 * eye(n)
        dr = lax.rsqrt(jnp.sum(ahat, axis=1, keepdims=True))
        xw = jnp.dot(x, W, preferred_element_type=f32)
        out = dr * jnp.dot(ahat, dr * xw, preferred_element_type=f32) + b
        return jnp.maximum(out, 0.0) if relu else out

    def augment(A, n):
        e = eye(n)
        at = A * (1.0 - e) + e
        return jnp.dot(at, at, preferred_element_type=f32) * (1.0 - e)

    def sel_mat_t(rank, n, kpad, k_val):
        rank_i = rank.astype(jnp.int32)
        cc = lax.broadcasted_iota(jnp.int32, (n, kpad), 1)
        return jnp.where((jnp.broadcast_to(rank_i, (n, kpad)) == cc)
                         & (cc < k_val), 1.0, 0.0)

    def topk(x, a_aug, w_col, n, kpad, k_val, valid_n):
        score = jnp.tanh(jnp.dot(x, w_col, preferred_element_type=f32))
        sm = jnp.where(col_iota(n) < valid_n, score, -2.0)
        rank = ranks(sm, n)
        rank_row = jnp.broadcast_to(rowvec(rank, n), (kpad, n)).astype(jnp.int32)
        rr = lax.broadcasted_iota(jnp.int32, (kpad, n), 0)
        p = jnp.where((rank_row == rr) & (rr < k_val), 1.0, 0.0)
        x_new = jnp.dot(p, x * score, preferred_element_type=f32)
        pa = jnp.dot(p, a_aug, preferred_element_type=f32)
        a_new = jnp.dot(pa, sel_mat_t(rank, n, kpad, k_val),
                        preferred_element_type=f32)
        return x_new, a_new, rank

    A = a_ref[0]
    x = gcn(A, x_ref[0], dw0_ref[...], db0_ref[...], True, SPAD[0])
    xs, adjs, rks = [x], [A], []
    for i in range(DEPTH):
        n, kpad = SPAD[i], SPAD[i + 1]
        valid = nv0 if i == 0 else kv[i - 1]
        a_aug = augment(A, n)
        x, A, rank = topk(x, a_aug, pw_ref[:, i:i + 1], n, kpad, kv[i], valid)
        x = gcn(A, x, dw_ref[i], db_ref[i], True, kpad)
        if i < DEPTH - 1:
            xs.append(x)
            adjs.append(A)
        rks.append(rank)

    for i in range(DEPTH):
        j = DEPTH - 1 - i
        n, kpad = SPAD[j], SPAD[j + 1]
        pt = sel_mat_t(rks[j], n, kpad, kv[j])
        up = jnp.dot(pt, x, preferred_element_type=f32)
        x = xs[j] + up
        if i < DEPTH - 1:
            x = gcn(adjs[j], x, uw_ref[i], ub_ref[i], True, n)
        else:
            x = gcn(adjs[j], x, uwl_ref[...], ubl_ref[...], False, n)

    n = SPAD[0]
    sc = x[:, TOTAL_LATENT - 1:TOTAL_LATENT]
    scm = jnp.where(col_iota(n) < nv0, sc, -1e30)
    srt = ranks(scm, n)
    rank_row = jnp.broadcast_to(rowvec(srt, n), (KP, n)).astype(jnp.int32)
    rr = lax.broadcasted_iota(jnp.int32, (KP, n), 0)
    p_even = jnp.where(rank_row == 2 * rr, 1.0, 0.0)
    p_odd = jnp.where(rank_row == 2 * rr + 1, 1.0, 0.0)
    xe = jnp.dot(p_even, x, preferred_element_type=f32)
    xo = jnp.dot(p_odd, x, preferred_element_type=f32)

    he = jnp.dot(xe, w1_ref[...], preferred_element_type=f32) + b1_ref[...]
    ho = jnp.dot(xo, w1_ref[...], preferred_element_type=f32) + b1_ref[...]
    hp = jnp.maximum(jnp.maximum(he, ho), 0.0)
    cols = jnp.concatenate([hp[j:j + LOUT, :] for j in range(KW2)], axis=1)
    h2 = jnp.maximum(
        jnp.dot(cols, w2_ref[...], preferred_element_type=f32) + b2_ref[...],
        0.0)
    out = bd_ref[...]
    for t in range(LOUT):
        out = out + jnp.dot(h2[t:t + 1, :], wd_ref[t], preferred_element_type=f32)
    o_ref[0] = jnp.maximum(out, 0.0)


def kernel(down_w0, down_b0, down_w, down_b, pool_w, up_w, up_b, up_w_last,
           up_b_last, w1, b1, w2, b2, wd, bd, x0, A0, x1, A1):
    pw = pool_w / jnp.sqrt(jnp.sum(pool_w * pool_w, axis=0, keepdims=True))
    pad = N_G0 - N_G1
    a_all = jnp.stack([A0, jnp.pad(A1, ((0, pad), (0, pad)))])
    x_all = jnp.stack([x0, jnp.pad(x1, ((0, pad), (0, 0)))])
    out = pl.pallas_call(
        _unet_kernel,
        out_shape=jax.ShapeDtypeStruct((2, 1, OUT_DIM), jnp.float32),
        grid=(2,),
        in_specs=[
            pl.BlockSpec((1, N_G0, N_G0), lambda g: (g, 0, 0)),
            pl.BlockSpec((1, N_G0, F_IN), lambda g: (g, 0, 0)),
            pl.BlockSpec((F_IN, H), lambda g: (0, 0)),
            pl.BlockSpec((1, H), lambda g: (0, 0)),
            pl.BlockSpec((DEPTH, H, H), lambda g: (0, 0, 0)),
            pl.BlockSpec((DEPTH, 1, H), lambda g: (0, 0, 0)),
            pl.BlockSpec((H, DEPTH), lambda g: (0, 0)),
            pl.BlockSpec((DEPTH - 1, H, H), lambda g: (0, 0, 0)),
            pl.BlockSpec((DEPTH - 1, 1, H), lambda g: (0, 0, 0)),
            pl.BlockSpec((H, PAD_LATENT), lambda g: (0, 0)),
            pl.BlockSpec((1, PAD_LATENT), lambda g: (0, 0)),
            pl.BlockSpec((PAD_LATENT, C1), lambda g: (0, 0)),
            pl.BlockSpec((1, C1), lambda g: (0, 0)),
            pl.BlockSpec((KW2 * C1, C2), lambda g: (0, 0)),
            pl.BlockSpec((1, C2), lambda g: (0, 0)),
            pl.BlockSpec((LOUT, C2, OUT_DIM), lambda g: (0, 0, 0)),
            pl.BlockSpec((1, OUT_DIM), lambda g: (0, 0)),
        ],
        out_specs=pl.BlockSpec((1, 1, OUT_DIM), lambda g: (g, 0, 0)),
        compiler_params=pltpu.CompilerParams(
            dimension_semantics=("parallel",)),
    )(a_all, x_all, down_w0, down_b0, down_w, down_b, pw, up_w, up_b,
      up_w_last, up_b_last, w1, b1, w2, b2, wd, bd)
    return out[:, 0, :]


# speedup vs baseline: 1.3334x; 1.3027x over previous
import math

import jax
import jax.numpy as jnp
from jax import lax
from jax.experimental import pallas as pl
from jax.experimental.pallas import tpu as pltpu

TOTAL_LATENT = 97
PAD_LATENT = 128
K_SORT = 30
C1, C2 = 16, 32
KW2 = 5
OUT_DIM = 10
F_IN = 8
H = 48
RATIOS = [0.9, 0.7, 0.6, 0.5]
DEPTH = len(RATIOS)
KP = (K_SORT - 2) // 2 + 1
LOUT = KP - KW2 + 1


def _ladder(n):
    out = [n]
    for r in RATIOS:
        out.append(int(math.ceil(r * out[-1])))
    return out


def _make_unet_kernel(lad):
    def unet_kernel(a_ref, x_ref, dw0_ref, db0_ref, dw_ref, db_ref, pw_ref,
                    uw_ref, ub_ref, uwl_ref, ubl_ref,
                    w1_ref, b1_ref, w2_ref, b2_ref, wd_ref, bd_ref, o_ref):
        f32 = jnp.float32
        _eyes, _ltm = {}, {}

        def eye(n):
            if n not in _eyes:
                r = lax.broadcasted_iota(jnp.int32, (n, n), 0)
                c = lax.broadcasted_iota(jnp.int32, (n, n), 1)
                _eyes[n] = jnp.where(r == c, 1.0, 0.0)
            return _eyes[n]

        def lt_mask(n):
            if n not in _ltm:
                r = lax.broadcasted_iota(jnp.int32, (n, n), 0)
                c = lax.broadcasted_iota(jnp.int32, (n, n), 1)
                _ltm[n] = c < r
            return _ltm[n]

        def rowvec(v_col, n):
            diag = eye(n) * jnp.broadcast_to(v_col, (n, n))
            return jnp.dot(jnp.ones((1, n), f32), diag,
                           preferred_element_type=f32)

        def ranks(s_col, n):
            s_row = jnp.broadcast_to(rowvec(s_col, n), (n, n))
            s_cb = jnp.broadcast_to(s_col, (n, n))
            before = jnp.where((s_row > s_cb) | ((s_row == s_cb) & lt_mask(n)),
                               1.0, 0.0)
            return jnp.sum(before, axis=1, keepdims=True)

        def gcn(A, x, W, b, relu, n):
            ahat = A + 2.0 * eye(n)
            dr = lax.rsqrt(jnp.sum(ahat, axis=1, keepdims=True))
            xw = jnp.dot(x, W, preferred_element_type=f32)
            out = dr * jnp.dot(ahat, dr * xw, preferred_element_type=f32) + b
            return jnp.maximum(out, 0.0) if relu else out

        def sel_mat_t(rank, n, k):
            rank_i = rank.astype(jnp.int32)
            cc = lax.broadcasted_iota(jnp.int32, (n, k), 1)
            return jnp.where(jnp.broadcast_to(rank_i, (n, k)) == cc, 1.0, 0.0)

        def topk_augmented(x, A, w_col, n, k):
            score = jnp.tanh(jnp.dot(x, w_col, preferred_element_type=f32))
            rank = ranks(score, n)
            rank_row = jnp.broadcast_to(rowvec(rank, n),
                                        (k, n)).astype(jnp.int32)
            rr = lax.broadcasted_iota(jnp.int32, (k, n), 0)
            p = jnp.where(rank_row == rr, 1.0, 0.0)
            x_new = jnp.dot(p, x * score, preferred_element_type=f32)
            e = eye(n)
            at = A * (1.0 - e) + e
            a_aug = jnp.dot(at, at, preferred_element_type=f32) * (1.0 - e)
            pa = jnp.dot(p, a_aug, preferred_element_type=f32)
            a_new = jnp.dot(pa, sel_mat_t(rank, n, k),
                            preferred_element_type=f32)
            return x_new, a_new, rank

        A = a_ref[...]
        x = gcn(A, x_ref[...], dw0_ref[...], db0_ref[...], True, lad[0])
        xs, adjs, rks = [x], [A], []
        for i in range(DEPTH):
            n, k = lad[i], lad[i + 1]
            x, A, rank = topk_augmented(x, A, pw_ref[:, i:i + 1], n, k)
            x = gcn(A, x, dw_ref[i], db_ref[i], True, k)
            if i < DEPTH - 1:
                xs.append(x)
                adjs.append(A)
            rks.append(rank)

        for i in range(DEPTH):
            j = DEPTH - 1 - i
            n, k = lad[j], lad[j + 1]
            pt = sel_mat_t(rks[j], n, k)
            up = jnp.dot(pt, x, preferred_element_type=f32)
            x = xs[j] + up
            if i < DEPTH - 1:
                x = gcn(adjs[j], x, uw_ref[i], ub_ref[i], True, n)
            else:
                x = gcn(adjs[j], x, uwl_ref[...], ubl_ref[...], False, n)

        n = lad[0]
        srt = ranks(x[:, TOTAL_LATENT - 1:TOTAL_LATENT], n)
        rank_row = jnp.broadcast_to(rowvec(srt, n), (KP, n)).astype(jnp.int32)
        rr = lax.broadcasted_iota(jnp.int32, (KP, n), 0)
        p_even = jnp.where(rank_row == 2 * rr, 1.0, 0.0)
        p_odd = jnp.where(rank_row == 2 * rr + 1, 1.0, 0.0)
        xe = jnp.dot(p_even, x, preferred_element_type=f32)
        xo = jnp.dot(p_odd, x, preferred_element_type=f32)

        he = jnp.dot(xe, w1_ref[...], preferred_element_type=f32) + b1_ref[...]
        ho = jnp.dot(xo, w1_ref[...], preferred_element_type=f32) + b1_ref[...]
        hp = jnp.maximum(jnp.maximum(he, ho), 0.0)
        cols = jnp.concatenate([hp[j:j + LOUT, :] for j in range(KW2)], axis=1)
        h2 = jnp.maximum(
            jnp.dot(cols, w2_ref[...], preferred_element_type=f32)
            + b2_ref[...], 0.0)
        out = bd_ref[...]
        for t in range(LOUT):
            out = out + jnp.dot(h2[t:t + 1, :], wd_ref[t],
                                preferred_element_type=f32)
        o_ref[...] = jnp.maximum(out, 0.0)

    return unet_kernel


def _graph_call(params, pw, x, A):
    k = _make_unet_kernel(_ladder(A.shape[0]))
    return pl.pallas_call(
        k, out_shape=jax.ShapeDtypeStruct((1, OUT_DIM), jnp.float32),
    )(A, x, params['down_w0'], params['down_b0'], params['down_w'],
      params['down_b'], pw, params['up_w'], params['up_b'],
      params['up_w_last'], params['up_b_last'], params['w1'], params['b1'],
      params['w2'], params['b2'], params['wd'], params['bd'])


def kernel(down_w0, down_b0, down_w, down_b, pool_w, up_w, up_b, up_w_last,
           up_b_last, w1, b1, w2, b2, wd, bd, x0, A0, x1, A1):
    params = {
        'down_w0': down_w0, 'down_b0': down_b0,
        'down_w': down_w, 'down_b': down_b,
        'up_w': up_w, 'up_b': up_b,
        'up_w_last': up_w_last, 'up_b_last': up_b_last,
        'w1': w1, 'b1': b1, 'w2': w2, 'b2': b2, 'wd': wd, 'bd': bd,
    }
    pw = pool_w / jnp.sqrt(jnp.sum(pool_w * pool_w, axis=0, keepdims=True))
    o0 = _graph_call(params, pw, x0, A0)
    o1 = _graph_call(params, pw, x1, A1)
    return jnp.concatenate([o0, o1], axis=0)
